# Initial kernel scaffold; baseline (speedup 1.0000x reference)
import jax
import jax.numpy as jnp
from jax.experimental import pallas as pl


def kernel(lprobs, scores, src_lengths, step):
    raise NotImplementedError("write your pallas kernel here")


# speedup vs baseline: 9.0510x; 9.0510x over previous
import functools

import jax
import jax.numpy as jnp
from jax import lax
from jax.experimental import pallas as pl
from jax.experimental.pallas import tpu as pltpu
from jax.experimental.pallas import tpu_sc as plsc

BSZ = 64
BEAM = 8
VOCAB = 100000
EOS = 2
ROW = BEAM * VOCAB
BLK = 800
NBLK = ROW // BLK
BPB = VOCAB // BLK
WIN = 40000
NWIN = ROW // WIN
VPB = BLK // 16
BPW = WIN // BLK
K = 16
NC, NS = 2, 16
NW = NC * NS
RPW = BSZ // NW


def _sc_body(lp_hbm, bias_hbm, src_hbm, step_hbm,
             outs_hbm, outi_hbm, outb_hbm,
             win0_v, win1_v, l1_v, l2_v, cand_v, l1c_v, l2c_v,
             bias_v, src_v, step_v, sel_v,
             outs_v, outi_v, outb_v,
             sem0, sem1, semg):
  cid = lax.axis_index("c")
  sid = lax.axis_index("s")
  wid = sid * NC + cid
  minf = jnp.float32(-jnp.inf)
  iota = lax.iota(jnp.int32, 16)
  lane0 = iota == 0

  pltpu.sync_copy(src_hbm, src_v)
  pltpu.sync_copy(step_hbm, step_v)
  step = step_v[pl.ds(0, 16)][0]

  def sread(ref, i):
    return plsc.load_gather(ref, [jnp.full((16,), i, jnp.int32)])[0]

  def sstore(ref, i, val):
    plsc.store_scatter(ref, [jnp.full((16,), i, jnp.int32)],
                       jnp.full((16,), val, ref.dtype), mask=lane0)

  def vmax50(load):
    accs = [load(0), load(1), load(2), load(3), load(4)]
    for v in range(5, VPB):
      accs[v % 5] = jnp.maximum(accs[v % 5], load(v))
    return jnp.maximum(jnp.maximum(jnp.maximum(accs[0], accs[1]),
                                   jnp.maximum(accs[2], accs[3])), accs[4])

  def ffs_scal(mask_vec):
    return plsc.all_reduce_ffs(mask_vec)[0]

  def argmax64(ref):
    vs = [ref[pl.ds(j * 16, 16)] for j in range(4)]
    m = jnp.maximum(jnp.maximum(vs[0], vs[1]), jnp.maximum(vs[2], vs[3]))
    gm = jnp.max(m)
    pos = jnp.int32(64)
    for j in reversed(range(4)):
      fj = ffs_scal(vs[j] == gm)
      pos = jnp.where(fj < 16, j * 16 + fj, pos)
    return gm, pos

  @pl.loop(0, RPW)
  def _row(ri):
    r = wid * RPW + ri
    row_base = r * ROW
    pltpu.sync_copy(bias_hbm.at[r], bias_v)
    src_len = sread(src_v, r)
    min_len = 0 * src_len + 1
    max_len = 2 * src_len + 10
    eos_ninf = jnp.logical_or(step < min_len, step > max_len)
    eos_zero = step == max_len

    sems = (sem0, sem1)

    bufs = (win0_v, win1_v)

    def copy_win(w):
      return pltpu.make_async_copy(
          lp_hbm.at[pl.ds(row_base + w * WIN, WIN)],
          bufs[w % 2], sems[w % 2])

    copy_win(0).start()
    for w in range(NWIN):
      if w + 1 < NWIN:
        copy_win(w + 1).start()
      copy_win(w).wait()
      buf = bufs[w % 2]

      @pl.loop(0, BPW)
      def _blk(b):
        base = b * BLK
        acc = vmax50(lambda v: buf[pl.ds(base + v * 16, 16)])
        g = w * BPW + b
        sstore(l1_v, g, jnp.max(acc) + sread(bias_v, g // BPB))

    for k in range(BEAM):
      pltpu.sync_copy(lp_hbm.at[pl.ds(row_base + k * VOCAB, BLK)],
                      win0_v.at[pl.ds(0, BLK)])
      v0 = win0_v[pl.ds(0, 16)]
      v0m = jnp.where(eos_ninf, minf,
                      jnp.where(eos_zero, jnp.float32(0.0), v0))
      v0 = jnp.where(iota == EOS, v0m, v0)

      def load_eos(v, _v0=v0):
        if v == 0:
          return _v0
        return win0_v[pl.ds(v * 16, 16)]

      acc = vmax50(load_eos)
      bias_vec = bias_v[pl.ds(0, 16)]
      sstore(l1_v, k * BPB, jnp.max(acc) + bias_vec[k])

    tail = l1_v[pl.ds(NBLK - 8, 16)]
    l1_v[pl.ds(NBLK - 8, 16)] = jnp.where(iota >= 8, minf, tail)

    @pl.loop(0, 63)
    def _l2(i):
      sstore(l2_v, i, jnp.max(l1_v[pl.ds(i * 16, 16)]))
    t2 = l2_v[pl.ds(48, 16)]
    l2_v[pl.ds(48, 16)] = jnp.where(iota == 15, minf, t2)

    @pl.loop(0, K)
    def _sel(t):
      gm, i2 = argmax64(l2_v)
      vi = l1_v[pl.ds(i2 * 16, 16)]
      e = ffs_scal(vi == gm)
      sstore(sel_v, t, i2 * 16 + e)
      vi2 = jnp.where(iota == e, minf, vi)
      l1_v[pl.ds(i2 * 16, 16)] = vi2
      sstore(l2_v, i2, jnp.max(vi2))

    @pl.loop(0, K)
    def _gstart(j):
      bid = sread(sel_v, j)
      pltpu.make_async_copy(
          lp_hbm.at[pl.ds(row_base + bid * BLK, BLK)],
          cand_v.at[pl.ds(j * BLK, BLK)], semg).start()

    @pl.loop(0, K)
    def _gwait(j):
      pltpu.make_async_copy(
          lp_hbm.at[pl.ds(row_base + j * BLK, BLK)],
          cand_v.at[pl.ds(j * BLK, BLK)], semg).wait()

    @pl.loop(0, K)
    def _fix(j):
      bid = sread(sel_v, j)
      b = sread(bias_v, bid // BPB)
      is_b0 = (bid % BPB) == 0

      @pl.loop(0, VPB)
      def _v(v):
        x = cand_v[pl.ds(j * BLK + v * 16, 16)]
        xm = jnp.where(eos_ninf, minf,
                       jnp.where(eos_zero, jnp.float32(0.0), x))
        x = jnp.where((iota == EOS) & is_b0 & (v == 0), xm, x)
        x = x + b
        cand_v[pl.ds(j * BLK + v * 16, 16)] = x
        sstore(l1c_v, j * VPB + v, jnp.max(x))

    @pl.loop(0, 50)
    def _l2c(i):
      sstore(l2c_v, i, jnp.max(l1c_v[pl.ds(i * 16, 16)]))
    t3 = l2c_v[pl.ds(48, 16)]
    l2c_v[pl.ds(48, 16)] = jnp.where(iota >= 2, minf, t3)

    @pl.loop(0, K)
    def _out(t):
      gm, i2 = argmax64(l2c_v)
      vi = l1c_v[pl.ds(i2 * 16, 16)]
      e = ffs_scal(vi == gm)
      q = i2 * 16 + e
      x = cand_v[pl.ds(q * 16, 16)]
      lane = ffs_scal(x == gm)
      bid = sread(sel_v, q // VPB)
      flat = bid * BLK + (q % VPB) * 16 + lane
      sstore(outs_v, t, gm)
      sstore(outb_v, t, flat // VOCAB)
      sstore(outi_v, t, flat % VOCAB)
      x2 = jnp.where(iota == lane, minf, x)
      cand_v[pl.ds(q * 16, 16)] = x2
      sstore(l1c_v, q, jnp.max(x2))
      vi2 = l1c_v[pl.ds(i2 * 16, 16)]
      sstore(l2c_v, i2, jnp.max(vi2))

    pltpu.sync_copy(outs_v, outs_hbm.at[r])
    pltpu.sync_copy(outi_v, outi_hbm.at[r])
    pltpu.sync_copy(outb_v, outb_hbm.at[r])


@functools.partial(
    pl.kernel,
    out_type=[
        jax.ShapeDtypeStruct((BSZ, K), jnp.float32),
        jax.ShapeDtypeStruct((BSZ, K), jnp.int32),
        jax.ShapeDtypeStruct((BSZ, K), jnp.int32),
    ],
    mesh=plsc.VectorSubcoreMesh(
        core_axis_name="c", subcore_axis_name="s",
        num_cores=NC, num_subcores=NS),
    compiler_params=pltpu.CompilerParams(needs_layout_passes=False),
    scratch_types=[
        pltpu.VMEM((WIN,), jnp.float32),
        pltpu.VMEM((WIN,), jnp.float32),
        pltpu.VMEM((NBLK + 8, ), jnp.float32),
        pltpu.VMEM((64,), jnp.float32),
        pltpu.VMEM((K * BLK,), jnp.float32),
        pltpu.VMEM((K * VPB,), jnp.float32),
        pltpu.VMEM((64,), jnp.float32),
        pltpu.VMEM((16,), jnp.float32),
        pltpu.VMEM((BSZ,), jnp.int32),
        pltpu.VMEM((16,), jnp.int32),
        pltpu.VMEM((16,), jnp.int32),
        pltpu.VMEM((16,), jnp.float32),
        pltpu.VMEM((16,), jnp.int32),
        pltpu.VMEM((16,), jnp.int32),
        pltpu.SemaphoreType.DMA,
        pltpu.SemaphoreType.DMA,
        pltpu.SemaphoreType.DMA,
    ],
)
def _sc_kernel(*args):
  _sc_body(*args)


def kernel(lprobs, scores, src_lengths, step):
  lp = lprobs.reshape(-1)
  step_i = jnp.asarray(step, jnp.int32)
  bias = lax.dynamic_index_in_dim(scores, step_i - 1, axis=2, keepdims=False)
  bias16 = jnp.concatenate(
      [bias.astype(jnp.float32), jnp.zeros((BSZ, 8), jnp.float32)], axis=1)
  src32 = src_lengths.astype(jnp.int32)
  step_arr = jnp.full((16,), step_i, jnp.int32)
  scores_buf, indices_buf, beams_buf = _sc_kernel(lp, bias16, src32, step_arr)
  return scores_buf, indices_buf, beams_buf


# speedup vs baseline: 17.4810x; 1.9314x over previous
import functools

import jax
import jax.numpy as jnp
from jax import lax
from jax.experimental import pallas as pl
from jax.experimental.pallas import tpu as pltpu
from jax.experimental.pallas import tpu_sc as plsc

BSZ = 64
BEAM = 8
VOCAB = 100000
EOS = 2
CW = 2560
NFW = VOCAB // CW
TAILC = VOCAB - NFW * CW
BLKC = 512
BPW = CW // BLKC
FBPB = VOCAB // BLKC
BPB = FBPB + 1
NBLK = BEAM * BPB
TAILV = TAILC // 16
VPB = BLKC // 16
K = 16
NC, NS = 2, 16
NW = NC * NS
RPW = BSZ // NW
L1N = 1600
L2N = 112


def _sc_body(lp_hbm, bias_hbm, src_hbm, step_hbm,
             outs_hbm, outi_hbm, outb_hbm,
             win0_v, win1_v, tail_v, l1_v, l2_v, cand_v, l1c_v, l2c_v,
             bias_v, src_v, step_v, sel_v,
             outs_v, outi_v, outb_v,
             sem0, sem1, semg):
  cid = lax.axis_index("c")
  sid = lax.axis_index("s")
  wid = sid * NC + cid
  minf = jnp.float32(-jnp.inf)
  iota = lax.iota(jnp.int32, 16)
  lane0 = iota == 0

  pltpu.sync_copy(src_hbm, src_v)
  pltpu.sync_copy(step_hbm, step_v)
  step = step_v[pl.ds(0, 16)][0]

  def sread(ref, i):
    return plsc.load_gather(ref, [jnp.full((16,), i, jnp.int32)])[0]

  def sstore(ref, i, val):
    plsc.store_scatter(ref, [jnp.full((16,), i, jnp.int32)],
                       jnp.full((16,), val, ref.dtype), mask=lane0)

  def ffs_scal(mask_vec):
    return plsc.all_reduce_ffs(mask_vec)[0]

  def vmaxn(load, n):
    na = min(4, n)
    accs = [load(v) for v in range(na)]
    for v in range(na, n):
      accs[v % na] = jnp.maximum(accs[v % na], load(v))
    while len(accs) > 1:
      accs = [jnp.maximum(accs[i], accs[i + 1])
              for i in range(0, len(accs) - 1, 2)] + (
                  [accs[-1]] if len(accs) % 2 else [])
    return accs[0]

  def argmax_ref(ref, nv):
    vs = [ref[pl.ds(j * 16, 16)] for j in range(nv)]
    m = vs[0]
    for j in range(1, nv):
      m = jnp.maximum(m, vs[j])
    gm = jnp.max(m)
    pos = jnp.int32(16 * nv)
    for j in reversed(range(nv)):
      fj = ffs_scal(vs[j] == gm)
      pos = jnp.where(fj < 16, j * 16 + fj, pos)
    return gm, pos

  @pl.loop(0, RPW)
  def _row(ri):
    r = wid * RPW + ri
    rb0 = r * BEAM
    pltpu.sync_copy(bias_hbm.at[r], bias_v)
    bias_vec = bias_v[pl.ds(0, 16)]
    src_len = sread(src_v, r)
    min_len = 0 * src_len + 1
    max_len = 2 * src_len + 10
    eos_ninf = jnp.logical_or(step < min_len, step > max_len)
    eos_zero = step == max_len

    def start_win(w, buf, sem):
      off = pl.multiple_of(w * CW, 128)
      pltpu.make_async_copy(
          lp_hbm.at[pl.ds(rb0, BEAM), pl.ds(off, CW)], buf, sem).start()

    def wait_win(w, buf, sem):
      off = pl.multiple_of(w * CW, 128)
      pltpu.make_async_copy(
          lp_hbm.at[pl.ds(rb0, BEAM), pl.ds(off, CW)], buf, sem).wait()

    def compute_win(w, buf):
      for beam_ in range(BEAM):
        @pl.loop(0, BPW)
        def _blk(b):
          acc = vmaxn(lambda v: buf[beam_, pl.ds(b * BLKC + v * 16, 16)],
                      VPB)
          sstore(l1_v, beam_ * BPB + w * BPW + b,
                 jnp.max(acc) + bias_vec[beam_])

    start_win(0, win0_v, sem0)
    start_win(1, win1_v, sem1)

    @pl.loop(0, (NFW - 1) // 2)
    def _wpair(p):
      w0 = 2 * p
      w1 = w0 + 1
      wait_win(w0, win0_v, sem0)
      compute_win(w0, win0_v)
      start_win(w0 + 2, win0_v, sem0)
      wait_win(w1, win1_v, sem1)
      compute_win(w1, win1_v)
      @pl.when(w1 + 2 < NFW)
      def _():
        start_win(w1 + 2, win1_v, sem1)

    wait_win(NFW - 1, win0_v, sem0)
    compute_win(NFW - 1, win0_v)

    pltpu.sync_copy(
        lp_hbm.at[pl.ds(rb0, BEAM), pl.ds(NFW * CW, TAILC)], tail_v)
    for beam_ in range(BEAM):
      acc = vmaxn(lambda v: tail_v[beam_, pl.ds(v * 16, 16)], TAILV)
      sstore(l1_v, beam_ * BPB + FBPB, jnp.max(acc) + bias_vec[beam_])

    pltpu.sync_copy(
        lp_hbm.at[pl.ds(rb0, BEAM), pl.ds(0, BLKC)],
        win0_v.at[pl.ds(0, BEAM), pl.ds(0, BLKC)])
    for beam_ in range(BEAM):
      v0 = win0_v[beam_, pl.ds(0, 16)]
      v0m = jnp.where(eos_ninf, minf,
                      jnp.where(eos_zero, jnp.float32(0.0), v0))
      v0 = jnp.where(iota == EOS, v0m, v0)

      def load_eos(v, _v0=v0, _b=beam_):
        if v == 0:
          return _v0
        return win0_v[_b, pl.ds(v * 16, 16)]

      acc = vmaxn(load_eos, VPB)
      sstore(l1_v, beam_ * BPB, jnp.max(acc) + bias_vec[beam_])

    l1_v[pl.ds(NBLK, 16)] = jnp.full((16,), minf, jnp.float32)
    l1_v[pl.ds(NBLK + 16, 16)] = jnp.full((16,), minf, jnp.float32)

    @pl.loop(0, L1N // 16)
    def _l2(i):
      sstore(l2_v, i, jnp.max(l1_v[pl.ds(i * 16, 16)]))
    t2 = l2_v[pl.ds(L2N - 16, 16)]
    l2_v[pl.ds(L2N - 16, 16)] = jnp.where(iota >= 4, minf, t2)

    @pl.loop(0, K)
    def _sel(t):
      gm, i2 = argmax_ref(l2_v, L2N // 16)
      vi = l1_v[pl.ds(i2 * 16, 16)]
      e = ffs_scal(vi == gm)
      sstore(sel_v, t, i2 * 16 + e)
      vi2 = jnp.where(iota == e, minf, vi)
      l1_v[pl.ds(i2 * 16, 16)] = vi2
      sstore(l2_v, i2, jnp.max(vi2))

    @pl.loop(0, K)
    def _gat(j):
      bid = sread(sel_v, j)
      beam = bid // BPB
      cb = bid % BPB
      dst_r = pl.multiple_of(j * 8, 8)
      src_r = pl.multiple_of(rb0 + 0 * beam, 8)

      @pl.when(cb < FBPB)
      def _full():
        off = pl.multiple_of(cb * BLKC, 128)
        pltpu.make_async_copy(
            lp_hbm.at[pl.ds(src_r, BEAM), pl.ds(off, BLKC)],
            cand_v.at[pl.ds(dst_r, BEAM), pl.ds(0, BLKC)], semg).start()
        pltpu.make_async_copy(
            lp_hbm.at[pl.ds(src_r, BEAM), pl.ds(off, BLKC)],
            cand_v.at[pl.ds(dst_r, BEAM), pl.ds(0, BLKC)], semg).wait()

      @pl.when(cb >= FBPB)
      def _tail():
        pltpu.make_async_copy(
            lp_hbm.at[pl.ds(src_r, BEAM), pl.ds(NFW * CW, TAILC)],
            tail_v, semg).start()
        pltpu.make_async_copy(
            lp_hbm.at[pl.ds(src_r, BEAM), pl.ds(NFW * CW, TAILC)],
            tail_v, semg).wait()
        row = j * 8 + beam
        for v in range(TAILV):
          cand_v[row, pl.ds(v * 16, 16)] = tail_v[beam, pl.ds(v * 16, 16)]

    @pl.loop(0, K)
    def _fix(j):
      bid = sread(sel_v, j)
      beam = bid // BPB
      cb = bid % BPB
      b = sread(bias_v, beam)
      is_b0 = cb == 0
      nv = jnp.where(cb < FBPB, VPB, TAILV)
      row = j * 8 + beam

      @pl.loop(0, VPB)
      def _v(v):
        x = cand_v[row, pl.ds(v * 16, 16)]
        xm = jnp.where(eos_ninf, minf,
                       jnp.where(eos_zero, jnp.float32(0.0), x))
        x = jnp.where((iota == EOS) & is_b0 & (v == 0), xm, x)
        x = x + b
        cand_v[row, pl.ds(v * 16, 16)] = x
        sstore(l1c_v, j * VPB + v, jnp.where(v < nv, jnp.max(x), minf))

    @pl.loop(0, K * VPB // 16)
    def _l2c(i):
      sstore(l2c_v, i, jnp.max(l1c_v[pl.ds(i * 16, 16)]))

    @pl.loop(0, K)
    def _out(t):
      gm, i2 = argmax_ref(l2c_v, K * VPB // 16 // 16)
      vi = l1c_v[pl.ds(i2 * 16, 16)]
      e = ffs_scal(vi == gm)
      q = i2 * 16 + e
      j = q // VPB
      v = q % VPB
      bid = sread(sel_v, j)
      beam = bid // BPB
      cb = bid % BPB
      row = j * 8 + beam
      x = cand_v[row, pl.ds(v * 16, 16)]
      lane = ffs_scal(x == gm)
      sstore(outs_v, t, gm)
      sstore(outb_v, t, beam)
      sstore(outi_v, t, cb * BLKC + v * 16 + lane)
      x2 = jnp.where(iota == lane, minf, x)
      cand_v[row, pl.ds(v * 16, 16)] = x2
      sstore(l1c_v, q, jnp.max(x2))
      vi2 = l1c_v[pl.ds(i2 * 16, 16)]
      sstore(l2c_v, i2, jnp.max(vi2))

    pltpu.sync_copy(outs_v, outs_hbm.at[r])
    pltpu.sync_copy(outi_v, outi_hbm.at[r])
    pltpu.sync_copy(outb_v, outb_hbm.at[r])


@functools.partial(
    pl.kernel,
    out_type=[
        jax.ShapeDtypeStruct((BSZ, K), jnp.float32),
        jax.ShapeDtypeStruct((BSZ, K), jnp.int32),
        jax.ShapeDtypeStruct((BSZ, K), jnp.int32),
    ],
    mesh=plsc.VectorSubcoreMesh(
        core_axis_name="c", subcore_axis_name="s",
        num_cores=NC, num_subcores=NS),
    compiler_params=pltpu.CompilerParams(needs_layout_passes=False),
    scratch_types=[
        pltpu.VMEM((BEAM, CW), jnp.float32),
        pltpu.VMEM((BEAM, CW), jnp.float32),
        pltpu.VMEM((BEAM, TAILC), jnp.float32),
        pltpu.VMEM((L1N,), jnp.float32),
        pltpu.VMEM((L2N,), jnp.float32),
        pltpu.VMEM((K * 8, BLKC), jnp.float32),
        pltpu.VMEM((K * VPB,), jnp.float32),
        pltpu.VMEM((K * VPB // 16,), jnp.float32),
        pltpu.VMEM((16,), jnp.float32),
        pltpu.VMEM((BSZ,), jnp.int32),
        pltpu.VMEM((16,), jnp.int32),
        pltpu.VMEM((16,), jnp.int32),
        pltpu.VMEM((16,), jnp.float32),
        pltpu.VMEM((16,), jnp.int32),
        pltpu.VMEM((16,), jnp.int32),
        pltpu.SemaphoreType.DMA,
        pltpu.SemaphoreType.DMA,
        pltpu.SemaphoreType.DMA,
    ],
)
def _sc_kernel(*args):
  _sc_body(*args)


def kernel(lprobs, scores, src_lengths, step):
  lp = lprobs.reshape(BSZ * BEAM, VOCAB)
  step_i = jnp.asarray(step, jnp.int32)
  bias = lax.dynamic_index_in_dim(scores, step_i - 1, axis=2, keepdims=False)
  bias16 = jnp.concatenate(
      [bias.astype(jnp.float32), jnp.zeros((BSZ, 8), jnp.float32)], axis=1)
  src32 = src_lengths.astype(jnp.int32)
  step_arr = jnp.full((16,), step_i, jnp.int32)
  scores_buf, indices_buf, beams_buf = _sc_kernel(lp, bias16, src32, step_arr)
  return scores_buf, indices_buf, beams_buf
